# baseline (speedup 1.0000x reference)
import functools

import jax
import jax.numpy as jnp
from jax import lax
from jax.experimental import pallas as pl
from jax.experimental.pallas import tpu as pltpu
from jax.experimental.pallas import tpu_sc as plsc

N = 10000
E = 320000
H = 128
B = 32

NC = 2
NS = 16
NW = NC * NS
ET = E // NW
DCHUNK = 80
DNCHUNK = ET // DCHUNK
CHUNK = 128
ETP = 10240
NCHUNK = ETP // CHUNK
EPAD = ETP * NW
NPAD = 10240
RPT = NPAD // NS
ZR = 128

_mesh = plsc.VectorSubcoreMesh(core_axis_name="c", subcore_axis_name="s",
                               num_cores=NC, num_subcores=NS)


def _lrelu(x):
    return jnp.where(x >= 0, x, 0.01 * x)


def _make_deg(width):
    @functools.partial(
        pl.kernel,
        out_type=jax.ShapeDtypeStruct((NC, NPAD, width), jnp.float32),
        mesh=_mesh,
        scratch_types=[
            pltpu.VMEM((DCHUNK,), jnp.int32),
            pltpu.VMEM((DCHUNK, width), jnp.float32),
            pltpu.VMEM((ZR, width), jnp.float32),
            pltpu.VMEM_SHARED((NPAD, width), jnp.float32),
        ],
    )
    def _deg_call(dst_hbm, out_hbm, idx_v, ones_v, zero_v, acc_sh):
        cid = lax.axis_index("c")
        sid = lax.axis_index("s")

        def fill_ones(i, carry):
            for j in range(width // 16):
                ones_v[i, pl.ds(j * 16, 16)] = jnp.ones((16,), jnp.float32)
            return carry

        lax.fori_loop(0, DCHUNK, fill_ones, 0)

        def fill_zero(i, carry):
            for j in range(width // 16):
                zero_v[i, pl.ds(j * 16, 16)] = jnp.zeros((16,), jnp.float32)
            return carry

        lax.fori_loop(0, ZR, fill_zero, 0)

        base = sid * RPT
        for j in range(RPT // ZR):
            pltpu.sync_copy(zero_v, acc_sh.at[pl.ds(base + j * ZR, ZR)])
        plsc.subcore_barrier()

        ebase = cid * (E // NC) + sid * ET

        def step(k, carry):
            pltpu.sync_copy(dst_hbm.at[pl.ds(ebase + k * DCHUNK, DCHUNK)], idx_v)
            pltpu.sync_copy(ones_v, acc_sh.at[idx_v], add=True)
            return carry

        lax.fori_loop(0, DNCHUNK, step, 0)
        plsc.subcore_barrier()
        pltpu.sync_copy(acc_sh.at[pl.ds(base, RPT)],
                        out_hbm.at[cid, pl.ds(base, RPT)])

    return _deg_call


_deg_call = _make_deg(128)


@functools.partial(
    pl.kernel,
    out_type=jax.ShapeDtypeStruct((NC, NPAD, H), jnp.float32),
    mesh=_mesh,
    scratch_types=[
        pltpu.VMEM((4, CHUNK), jnp.int32),
        pltpu.VMEM((4, CHUNK), jnp.int32),
        pltpu.VMEM((2, CHUNK, H), jnp.float32),
        pltpu.VMEM_SHARED((NPAD, H), jnp.float32),
        pltpu.SemaphoreType.DMA,
        pltpu.SemaphoreType.DMA,
        pltpu.SemaphoreType.DMA,
        pltpu.SemaphoreType.DMA,
        pltpu.SemaphoreType.DMA,
        pltpu.SemaphoreType.DMA,
    ],
)
def _seg_call(h_hbm, src_hbm, dst_hbm, out_hbm, src_v, dst_v, msgs_v,
              acc_sh, gsem0, gsem1, isem0, isem1, ssem0, ssem1):
    cid = lax.axis_index("c")
    sid = lax.axis_index("s")
    wid = cid * NS + sid

    def fill_zero(i, carry):
        for j in range(H // 16):
            msgs_v[0, i, pl.ds(j * 16, 16)] = jnp.zeros((16,), jnp.float32)
        return carry

    lax.fori_loop(0, CHUNK, fill_zero, 0)
    base = sid * RPT
    for j in range(RPT // CHUNK):
        pltpu.sync_copy(msgs_v.at[0], acc_sh.at[pl.ds(base + j * CHUNK, CHUNK)])
    plsc.subcore_barrier()

    isems = (isem0, isem1)
    gsems = (gsem0, gsem1)

    def idx_issue(k, isem):
        pltpu.async_copy(src_hbm.at[wid, k], src_v.at[lax.rem(k, 4)], isem)
        pltpu.async_copy(dst_hbm.at[wid, k], dst_v.at[lax.rem(k, 4)], isem)

    def idx_drain(isem):
        pltpu.make_async_copy(src_hbm.at[0, 0], src_v.at[0], isem).wait()
        pltpu.make_async_copy(src_hbm.at[0, 0], src_v.at[0], isem).wait()

    def gat_issue(k, b, gsem):
        pltpu.async_copy(h_hbm.at[src_v.at[lax.rem(k, 4)]], msgs_v.at[b], gsem)

    def gat_drain(b, gsem):
        pltpu.make_async_copy(h_hbm.at[src_v.at[0]], msgs_v.at[b], gsem).wait()

    ssems = (ssem0, ssem1)

    def scat_issue(k, b, ssem):
        pltpu.async_copy(msgs_v.at[b], acc_sh.at[dst_v.at[lax.rem(k, 4)]],
                         ssem, add=True)

    def scat_drain(b, ssem):
        pltpu.make_async_copy(h_hbm.at[src_v.at[0]], msgs_v.at[b], ssem).wait()

    idx_issue(0, isems[0])
    idx_issue(1, isems[1])
    idx_drain(isems[0])
    gat_issue(0, 0, gsems[0])
    idx_issue(2, isems[0])

    def body(k, carry):
        even = lax.rem(k, 2) == 0

        def stage(par):
            @pl.when(k + 1 < NCHUNK)
            def _():
                idx_drain(isems[1 - par])
                @pl.when(k >= 1)
                def _():
                    scat_drain(1 - par, ssems[1 - par])
                gat_issue(k + 1, 1 - par, gsems[1 - par])

            @pl.when(k + 3 < NCHUNK)
            def _():
                idx_issue(k + 3, isems[1 - par])

            gat_drain(par, gsems[par])
            scat_issue(k, par, ssems[par])

        @pl.when(even)
        def _():
            stage(0)

        @pl.when(jnp.logical_not(even))
        def _():
            stage(1)

        return carry

    lax.fori_loop(0, NCHUNK, body, 0)
    scat_drain(0, ssems[0])
    scat_drain(1, ssems[1])
    plsc.subcore_barrier()
    pltpu.sync_copy(acc_sh.at[pl.ds(base, RPT)], out_hbm.at[cid, pl.ds(base, RPT)])


R = 400
GRID = N // R


def _row_spec(width):
    return pl.BlockSpec((R, width), lambda i: (i, 0))


def _full_spec(a, b):
    return pl.BlockSpec((a, b), lambda i: (0, 0))


def _enc_body(desc, tweet, nump, catp, wd, wt, wn, wc, bd, bt, bn, bc,
              wid_, wit_, win_, wic_, bin_, out):
    d = _lrelu(jnp.dot(desc[...], wd[...], preferred_element_type=jnp.float32) + bd[...])
    t = _lrelu(jnp.dot(tweet[...], wt[...], preferred_element_type=jnp.float32) + bt[...])
    n = _lrelu(jnp.dot(nump[...], wn[...], preferred_element_type=jnp.float32) + bn[...])
    c = _lrelu(jnp.dot(catp[...], wc[...], preferred_element_type=jnp.float32) + bc[...])
    acc = (jnp.dot(d, wid_[...], preferred_element_type=jnp.float32)
           + jnp.dot(t, wit_[...], preferred_element_type=jnp.float32)
           + jnp.dot(n, win_[...], preferred_element_type=jnp.float32)
           + jnp.dot(c, wic_[...], preferred_element_type=jnp.float32))
    out[...] = _lrelu(acc + bin_[...])


def _dis(deg0, deg1):
    return lax.rsqrt(1.0 + deg0[:, 0:1] + deg1[:, 0:1])


def _h1_body(x0, deg0, deg1, w1t, h1):
    dis = _dis(deg0[...], deg1[...])
    h1[...] = jnp.dot(x0[...], w1t[...], preferred_element_type=jnp.float32) * dis


def _mid_body(p0, p1, hprev, deg0, deg1, bprev, w2t, hnext):
    dis = _dis(deg0[...], deg1[...])
    x1 = dis * (p0[...] + p1[...] + hprev[...]) + bprev[...]
    hnext[...] = jnp.dot(x1, w2t[...], preferred_element_type=jnp.float32) * dis


def _tail_body(p0, p1, hprev, deg0, deg1, bprev, wot, bo, wht, bh, out):
    dis = _dis(deg0[...], deg1[...])
    x2 = dis * (p0[...] + p1[...] + hprev[...]) + bprev[...]
    y = _lrelu(jnp.dot(x2, wot[...], preferred_element_type=jnp.float32) + bo[...])
    out[...] = jnp.dot(y, wht[...], preferred_element_type=jnp.float32) + bh[...]


def kernel(description, tweet, num_prop, cat_prop, edge_index, W_desc, b_desc,
           W_tweet, b_tweet, W_num, b_num, W_cat, b_cat, W_in, b_in, gcn1_W,
           gcn1_b, gcn2_W, gcn2_b, W_out, b_out, W_head, b_head):
    f32 = jnp.float32
    src = edge_index[0].astype(jnp.int32)
    dst = edge_index[1].astype(jnp.int32)

    deg_parts = _deg_call(dst)
    deg0, deg1 = deg_parts[0, :N, :16], deg_parts[1, :N, :16]

    enc = pl.pallas_call(
        _enc_body,
        grid=(GRID,),
        in_specs=[
            _row_spec(768), _row_spec(768), _row_spec(5), _row_spec(3),
            _full_spec(768, B), _full_spec(768, B), _full_spec(5, B), _full_spec(3, B),
            _full_spec(1, B), _full_spec(1, B), _full_spec(1, B), _full_spec(1, B),
            _full_spec(B, H), _full_spec(B, H), _full_spec(B, H), _full_spec(B, H),
            _full_spec(1, H),
        ],
        out_specs=_row_spec(H),
        out_shape=jax.ShapeDtypeStruct((N, H), f32),
    )
    x0 = enc(description, tweet, num_prop, cat_prop,
             W_desc.T, W_tweet.T, W_num.T, W_cat.T,
             b_desc.reshape(1, B), b_tweet.reshape(1, B),
             b_num.reshape(1, B), b_cat.reshape(1, B),
             W_in[:, 0:B].T, W_in[:, B:2 * B].T, W_in[:, 2 * B:3 * B].T,
             W_in[:, 3 * B:4 * B].T, b_in.reshape(1, H))

    h1 = pl.pallas_call(
        _h1_body,
        grid=(GRID,),
        in_specs=[_row_spec(H), _row_spec(16), _row_spec(16), _full_spec(H, H)],
        out_specs=_row_spec(H),
        out_shape=jax.ShapeDtypeStruct((N, H), f32),
    )(x0, deg0, deg1, gcn1_W.T)

    npad_e = EPAD - E
    src3 = jnp.concatenate(
        [src, jnp.zeros((npad_e,), jnp.int32)]).reshape(NW, NCHUNK, CHUNK)
    dst3 = jnp.concatenate(
        [dst, jnp.full((npad_e,), NPAD - 8, jnp.int32)]).reshape(NW, NCHUNK, CHUNK)
    s1 = _seg_call(h1, src3, dst3)

    h2 = pl.pallas_call(
        _mid_body,
        grid=(GRID,),
        in_specs=[_row_spec(H), _row_spec(H), _row_spec(H), _row_spec(16),
                  _row_spec(16), _full_spec(1, H), _full_spec(H, H)],
        out_specs=_row_spec(H),
        out_shape=jax.ShapeDtypeStruct((N, H), f32),
    )(s1[0, :N], s1[1, :N], h1, deg0, deg1, gcn1_b.reshape(1, H), gcn2_W.T)

    s2 = _seg_call(h2, src3, dst3)

    out = pl.pallas_call(
        _tail_body,
        grid=(GRID,),
        in_specs=[_row_spec(H), _row_spec(H), _row_spec(H), _row_spec(16),
                  _row_spec(16), _full_spec(1, H), _full_spec(H, H),
                  _full_spec(1, H), _full_spec(H, 2), _full_spec(1, 2)],
        out_specs=_row_spec(2),
        out_shape=jax.ShapeDtypeStruct((N, 2), f32),
    )(s2[0, :N], s2[1, :N], h2, deg0, deg1, gcn2_b.reshape(1, H), W_out.T,
      b_out.reshape(1, H), W_head.T, b_head.reshape(1, 2))

    return out


# speedup vs baseline: 2.5043x; 2.5043x over previous
import functools

import jax
import jax.numpy as jnp
from jax import lax
from jax.experimental import pallas as pl
from jax.experimental.pallas import tpu as pltpu
from jax.experimental.pallas import tpu_sc as plsc

N = 10000
E = 320000
H = 128
B = 32

NC = 2
NS = 16
NW = NC * NS
ET = E // NW
DCHUNK = 80
DNCHUNK = ET // DCHUNK
CHUNK = 128
ETP = 10240
NCHUNK = ETP // CHUNK
EPAD = ETP * NW
NPAD = 10240
RPT = NPAD // NS
ZR = 128

_mesh = plsc.VectorSubcoreMesh(core_axis_name="c", subcore_axis_name="s",
                               num_cores=NC, num_subcores=NS)


def _lrelu(x):
    return jnp.where(x >= 0, x, 0.01 * x)


def _make_deg(width):
    @functools.partial(
        pl.kernel,
        out_type=jax.ShapeDtypeStruct((NC, NPAD, width), jnp.float32),
        mesh=_mesh,
        scratch_types=[
            pltpu.VMEM((DCHUNK,), jnp.int32),
            pltpu.VMEM((DCHUNK, width), jnp.float32),
            pltpu.VMEM((ZR, width), jnp.float32),
            pltpu.VMEM_SHARED((NPAD, width), jnp.float32),
        ],
    )
    def _deg_call(dst_hbm, out_hbm, idx_v, ones_v, zero_v, acc_sh):
        cid = lax.axis_index("c")
        sid = lax.axis_index("s")

        def fill_ones(i, carry):
            for j in range(width // 16):
                ones_v[i, pl.ds(j * 16, 16)] = jnp.ones((16,), jnp.float32)
            return carry

        lax.fori_loop(0, DCHUNK, fill_ones, 0)

        def fill_zero(i, carry):
            for j in range(width // 16):
                zero_v[i, pl.ds(j * 16, 16)] = jnp.zeros((16,), jnp.float32)
            return carry

        lax.fori_loop(0, ZR, fill_zero, 0)

        base = sid * RPT
        for j in range(RPT // ZR):
            pltpu.sync_copy(zero_v, acc_sh.at[pl.ds(base + j * ZR, ZR)])
        plsc.subcore_barrier()

        ebase = cid * (E // NC) + sid * ET

        def step(k, carry):
            pltpu.sync_copy(dst_hbm.at[pl.ds(ebase + k * DCHUNK, DCHUNK)], idx_v)
            pltpu.sync_copy(ones_v, acc_sh.at[idx_v], add=True)
            return carry

        lax.fori_loop(0, DNCHUNK, step, 0)
        plsc.subcore_barrier()
        pltpu.sync_copy(acc_sh.at[pl.ds(base, RPT)],
                        out_hbm.at[cid, pl.ds(base, RPT)])

    return _deg_call


_deg_call = _make_deg(128)


@functools.partial(
    pl.kernel,
    out_type=jax.ShapeDtypeStruct((NC, NPAD, H), jnp.float32),
    mesh=_mesh,
    scratch_types=[
        pltpu.VMEM((4, CHUNK), jnp.int32),
        pltpu.VMEM((4, CHUNK), jnp.int32),
        pltpu.VMEM((2, CHUNK, H), jnp.float32),
        pltpu.VMEM_SHARED((NPAD, H), jnp.float32),
        pltpu.SemaphoreType.DMA,
        pltpu.SemaphoreType.DMA,
        pltpu.SemaphoreType.DMA,
        pltpu.SemaphoreType.DMA,
        pltpu.SemaphoreType.DMA,
        pltpu.SemaphoreType.DMA,
    ],
)
def _seg_call(h_hbm, src_hbm, dst_hbm, out_hbm, src_v, dst_v, msgs_v,
              acc_sh, gsem0, gsem1, isem0, isem1, ssem0, ssem1):
    cid = lax.axis_index("c")
    sid = lax.axis_index("s")
    wid = cid * NS + sid

    def fill_zero(i, carry):
        for j in range(H // 16):
            msgs_v[0, i, pl.ds(j * 16, 16)] = jnp.zeros((16,), jnp.float32)
        return carry

    lax.fori_loop(0, CHUNK, fill_zero, 0)
    base = sid * RPT
    for j in range(RPT // CHUNK):
        pltpu.sync_copy(msgs_v.at[0], acc_sh.at[pl.ds(base + j * CHUNK, CHUNK)])
    plsc.subcore_barrier()

    isems = (isem0, isem1)
    gsems = (gsem0, gsem1)

    def idx_issue(k, isem):
        pltpu.async_copy(src_hbm.at[wid, k], src_v.at[lax.rem(k, 4)], isem)
        pltpu.async_copy(dst_hbm.at[wid, k], dst_v.at[lax.rem(k, 4)], isem)

    def idx_drain(isem):
        pltpu.make_async_copy(src_hbm.at[0, 0], src_v.at[0], isem).wait()
        pltpu.make_async_copy(src_hbm.at[0, 0], src_v.at[0], isem).wait()

    def gat_issue(k, b, gsem):
        pltpu.async_copy(h_hbm.at[src_v.at[lax.rem(k, 4)]], msgs_v.at[b], gsem)

    def gat_drain(b, gsem):
        pltpu.make_async_copy(h_hbm.at[src_v.at[0]], msgs_v.at[b], gsem).wait()

    ssems = (ssem0, ssem1)

    def scat_issue(k, b, ssem):
        pltpu.async_copy(msgs_v.at[b], acc_sh.at[dst_v.at[lax.rem(k, 4)]],
                         ssem, add=True)

    def scat_drain(b, ssem):
        pltpu.make_async_copy(h_hbm.at[src_v.at[0]], msgs_v.at[b], ssem).wait()

    idx_issue(0, isems[0])
    idx_issue(1, isems[1])
    idx_drain(isems[0])
    gat_issue(0, 0, gsems[0])
    idx_issue(2, isems[0])

    def body(k, carry):
        even = lax.rem(k, 2) == 0

        def stage(par):
            @pl.when(k + 1 < NCHUNK)
            def _():
                idx_drain(isems[1 - par])
                @pl.when(k >= 1)
                def _():
                    scat_drain(1 - par, ssems[1 - par])
                gat_issue(k + 1, 1 - par, gsems[1 - par])

            @pl.when(k + 3 < NCHUNK)
            def _():
                idx_issue(k + 3, isems[1 - par])

            gat_drain(par, gsems[par])
            scat_issue(k, par, ssems[par])

        @pl.when(even)
        def _():
            stage(0)

        @pl.when(jnp.logical_not(even))
        def _():
            stage(1)

        return carry

    lax.fori_loop(0, NCHUNK, body, 0)
    scat_drain(0, ssems[0])
    scat_drain(1, ssems[1])
    plsc.subcore_barrier()
    pltpu.sync_copy(acc_sh.at[pl.ds(base, RPT)], out_hbm.at[cid, pl.ds(base, RPT)])


R = 400
GRID = N // R


def _row_spec(width):
    return pl.BlockSpec((R, width), lambda i: (i, 0))


def _full_spec(a, b):
    return pl.BlockSpec((a, b), lambda i: (0, 0))


def _enc_body(desc, tweet, nump, catp, wd, wt, wn, wc, bd, bt, bn, bc,
              wid_, wit_, win_, wic_, bin_, out):
    d = _lrelu(jnp.dot(desc[...], wd[...], preferred_element_type=jnp.float32) + bd[...])
    t = _lrelu(jnp.dot(tweet[...], wt[...], preferred_element_type=jnp.float32) + bt[...])
    n = _lrelu(jnp.dot(nump[...], wn[...], preferred_element_type=jnp.float32) + bn[...])
    c = _lrelu(jnp.dot(catp[...], wc[...], preferred_element_type=jnp.float32) + bc[...])
    acc = (jnp.dot(d, wid_[...], preferred_element_type=jnp.float32)
           + jnp.dot(t, wit_[...], preferred_element_type=jnp.float32)
           + jnp.dot(n, win_[...], preferred_element_type=jnp.float32)
           + jnp.dot(c, wic_[...], preferred_element_type=jnp.float32))
    out[...] = _lrelu(acc + bin_[...])


def _dis(deg0, deg1):
    return lax.rsqrt(1.0 + deg0[:, 0:1] + deg1[:, 0:1])


def _h1_body(x0, deg0, deg1, w1t, h1):
    dis = _dis(deg0[...], deg1[...])
    h1[...] = jnp.dot(x0[...], w1t[...], preferred_element_type=jnp.float32) * dis


def _mid_body(p0, p1, hprev, deg0, deg1, bprev, w2t, hnext):
    dis = _dis(deg0[...], deg1[...])
    x1 = dis * (p0[...] + p1[...] + hprev[...]) + bprev[...]
    hnext[...] = jnp.dot(x1, w2t[...], preferred_element_type=jnp.float32) * dis


def _tail_body(p0, p1, hprev, deg0, deg1, bprev, wot, bo, wht, bh, out):
    dis = _dis(deg0[...], deg1[...])
    x2 = dis * (p0[...] + p1[...] + hprev[...]) + bprev[...]
    y = _lrelu(jnp.dot(x2, wot[...], preferred_element_type=jnp.float32) + bo[...])
    out[...] = jnp.dot(y, wht[...], preferred_element_type=jnp.float32) + bh[...]


def kernel(description, tweet, num_prop, cat_prop, edge_index, W_desc, b_desc,
           W_tweet, b_tweet, W_num, b_num, W_cat, b_cat, W_in, b_in, gcn1_W,
           gcn1_b, gcn2_W, gcn2_b, W_out, b_out, W_head, b_head):
    f32 = jnp.float32
    src = edge_index[0].astype(jnp.int32)
    dst = edge_index[1].astype(jnp.int32)

    deg_parts = _deg_call(dst)
    deg0, deg1 = deg_parts[0, :N, :16], deg_parts[1, :N, :16]

    enc = pl.pallas_call(
        _enc_body,
        grid=(GRID,),
        in_specs=[
            _row_spec(768), _row_spec(768), _row_spec(5), _row_spec(3),
            _full_spec(768, B), _full_spec(768, B), _full_spec(5, B), _full_spec(3, B),
            _full_spec(1, B), _full_spec(1, B), _full_spec(1, B), _full_spec(1, B),
            _full_spec(B, H), _full_spec(B, H), _full_spec(B, H), _full_spec(B, H),
            _full_spec(1, H),
        ],
        out_specs=_row_spec(H),
        out_shape=jax.ShapeDtypeStruct((N, H), f32),
    )
    x0 = enc(description, tweet, num_prop, cat_prop,
             W_desc.T, W_tweet.T, W_num.T, W_cat.T,
             b_desc.reshape(1, B), b_tweet.reshape(1, B),
             b_num.reshape(1, B), b_cat.reshape(1, B),
             W_in[:, 0:B].T, W_in[:, B:2 * B].T, W_in[:, 2 * B:3 * B].T,
             W_in[:, 3 * B:4 * B].T, b_in.reshape(1, H))

    h1 = pl.pallas_call(
        _h1_body,
        grid=(GRID,),
        in_specs=[_row_spec(H), _row_spec(16), _row_spec(16), _full_spec(H, H)],
        out_specs=_row_spec(H),
        out_shape=jax.ShapeDtypeStruct((N, H), f32),
    )(x0, deg0, deg1, gcn1_W.T)

    npad_e = EPAD - E
    spread = jnp.arange(npad_e, dtype=jnp.int32) % (NPAD - N)
    src3 = jnp.concatenate(
        [src, spread]).reshape(NW, NCHUNK, CHUNK)
    dst3 = jnp.concatenate(
        [dst, N + spread]).reshape(NW, NCHUNK, CHUNK)
    s1 = _seg_call(h1, src3, dst3)

    h2 = pl.pallas_call(
        _mid_body,
        grid=(GRID,),
        in_specs=[_row_spec(H), _row_spec(H), _row_spec(H), _row_spec(16),
                  _row_spec(16), _full_spec(1, H), _full_spec(H, H)],
        out_specs=_row_spec(H),
        out_shape=jax.ShapeDtypeStruct((N, H), f32),
    )(s1[0, :N], s1[1, :N], h1, deg0, deg1, gcn1_b.reshape(1, H), gcn2_W.T)

    s2 = _seg_call(h2, src3, dst3)

    out = pl.pallas_call(
        _tail_body,
        grid=(GRID,),
        in_specs=[_row_spec(H), _row_spec(H), _row_spec(H), _row_spec(16),
                  _row_spec(16), _full_spec(1, H), _full_spec(H, H),
                  _full_spec(1, H), _full_spec(H, 2), _full_spec(1, 2)],
        out_specs=_row_spec(2),
        out_shape=jax.ShapeDtypeStruct((N, 2), f32),
    )(s2[0, :N], s2[1, :N], h2, deg0, deg1, gcn2_b.reshape(1, H), W_out.T,
      b_out.reshape(1, H), W_head.T, b_head.reshape(1, 2))

    return out


# speedup vs baseline: 2.8129x; 1.1232x over previous
import functools

import jax
import jax.numpy as jnp
from jax import lax
from jax.experimental import pallas as pl
from jax.experimental.pallas import tpu as pltpu
from jax.experimental.pallas import tpu_sc as plsc

N = 10000
E = 320000
H = 128
B = 32

NC = 2
NS = 16
NW = NC * NS
ET = E // NW
DCHUNK = 80
DNCHUNK = ET // DCHUNK
CHUNK = 128
ETP = 10240
NCHUNK = ETP // CHUNK
EPAD = ETP * NW
NPAD = 10240
RPT = NPAD // NS
ZR = 128

_mesh = plsc.VectorSubcoreMesh(core_axis_name="c", subcore_axis_name="s",
                               num_cores=NC, num_subcores=NS)


def _lrelu(x):
    return jnp.where(x >= 0, x, 0.01 * x)


@functools.partial(
    pl.kernel,
    out_type=jax.ShapeDtypeStruct((NC, NPAD, H), jnp.float32),
    mesh=_mesh,
    scratch_types=[
        pltpu.VMEM((4, CHUNK), jnp.int32),
        pltpu.VMEM((CHUNK, H), jnp.float32),
        pltpu.VMEM_SHARED((NPAD, H), jnp.float32),
        pltpu.SemaphoreType.DMA,
        pltpu.SemaphoreType.DMA,
        pltpu.SemaphoreType.DMA,
        pltpu.SemaphoreType.DMA,
    ],
)
def _deg_call(dst_hbm, out_hbm, dst_v, ones_v, acc_sh, isem0, isem1,
              ssem0, ssem1):
    cid = lax.axis_index("c")
    sid = lax.axis_index("s")
    wid = cid * NS + sid

    def fill(i, val):
        def body(j, carry):
            ones_v[j, pl.ds(i * 16, 16)] = jnp.full((16,), val, jnp.float32)
            return carry
        return body

    for i in range(H // 16):
        lax.fori_loop(0, CHUNK, fill(i, 0.0), 0)
    base = sid * RPT
    for j in range(RPT // CHUNK):
        pltpu.sync_copy(ones_v, acc_sh.at[pl.ds(base + j * CHUNK, CHUNK)])
    for i in range(H // 16):
        lax.fori_loop(0, CHUNK, fill(i, 1.0), 0)
    plsc.subcore_barrier()

    isems = (isem0, isem1)
    ssems = (ssem0, ssem1)

    def idx_issue(k, isem):
        pltpu.async_copy(dst_hbm.at[wid, k], dst_v.at[lax.rem(k, 4)], isem)

    def idx_drain(isem):
        pltpu.make_async_copy(dst_hbm.at[0, 0], dst_v.at[0], isem).wait()

    def scat_issue(k, ssem):
        pltpu.async_copy(ones_v, acc_sh.at[dst_v.at[lax.rem(k, 4)]], ssem,
                         add=True)

    def scat_drain(ssem):
        pltpu.make_async_copy(dst_hbm.at[0], ones_v, ssem).wait()

    idx_issue(0, isems[0])
    idx_issue(1, isems[1])

    def body(k, carry):
        even = lax.rem(k, 2) == 0

        def stage(par):
            @pl.when(k + 2 < NCHUNK)
            def _():
                idx_issue(k + 2, isems[par])

            idx_drain(isems[par])

            @pl.when(k >= 2)
            def _():
                scat_drain(ssems[par])

            scat_issue(k, ssems[par])

        @pl.when(even)
        def _():
            stage(0)

        @pl.when(jnp.logical_not(even))
        def _():
            stage(1)

        return carry

    lax.fori_loop(0, NCHUNK, body, 0)
    scat_drain(ssems[0])
    scat_drain(ssems[1])
    plsc.subcore_barrier()
    pltpu.sync_copy(acc_sh.at[pl.ds(base, RPT)],
                    out_hbm.at[cid, pl.ds(base, RPT)])


@functools.partial(
    pl.kernel,
    out_type=jax.ShapeDtypeStruct((NC, NPAD, H), jnp.float32),
    mesh=_mesh,
    scratch_types=[
        pltpu.VMEM((4, CHUNK), jnp.int32),
        pltpu.VMEM((4, CHUNK), jnp.int32),
        pltpu.VMEM((2, CHUNK, H), jnp.float32),
        pltpu.VMEM_SHARED((NPAD, H), jnp.float32),
        pltpu.SemaphoreType.DMA,
        pltpu.SemaphoreType.DMA,
        pltpu.SemaphoreType.DMA,
        pltpu.SemaphoreType.DMA,
        pltpu.SemaphoreType.DMA,
        pltpu.SemaphoreType.DMA,
    ],
)
def _seg_call(h_hbm, src_hbm, dst_hbm, out_hbm, src_v, dst_v, msgs_v,
              acc_sh, gsem0, gsem1, isem0, isem1, ssem0, ssem1):
    cid = lax.axis_index("c")
    sid = lax.axis_index("s")
    wid = cid * NS + sid

    def fill_zero(i, carry):
        for j in range(H // 16):
            msgs_v[0, i, pl.ds(j * 16, 16)] = jnp.zeros((16,), jnp.float32)
        return carry

    lax.fori_loop(0, CHUNK, fill_zero, 0)
    base = sid * RPT
    for j in range(RPT // CHUNK):
        pltpu.sync_copy(msgs_v.at[0], acc_sh.at[pl.ds(base + j * CHUNK, CHUNK)])
    plsc.subcore_barrier()

    isems = (isem0, isem1)
    gsems = (gsem0, gsem1)

    def idx_issue(k, isem):
        pltpu.async_copy(src_hbm.at[wid, k], src_v.at[lax.rem(k, 4)], isem)
        pltpu.async_copy(dst_hbm.at[wid, k], dst_v.at[lax.rem(k, 4)], isem)

    def idx_drain(isem):
        pltpu.make_async_copy(src_hbm.at[0, 0], src_v.at[0], isem).wait()
        pltpu.make_async_copy(src_hbm.at[0, 0], src_v.at[0], isem).wait()

    def gat_issue(k, b, gsem):
        pltpu.async_copy(h_hbm.at[src_v.at[lax.rem(k, 4)]], msgs_v.at[b], gsem)

    def gat_drain(b, gsem):
        pltpu.make_async_copy(h_hbm.at[src_v.at[0]], msgs_v.at[b], gsem).wait()

    ssems = (ssem0, ssem1)

    def scat_issue(k, b, ssem):
        pltpu.async_copy(msgs_v.at[b], acc_sh.at[dst_v.at[lax.rem(k, 4)]],
                         ssem, add=True)

    def scat_drain(b, ssem):
        pltpu.make_async_copy(h_hbm.at[src_v.at[0]], msgs_v.at[b], ssem).wait()

    idx_issue(0, isems[0])
    idx_issue(1, isems[1])
    idx_drain(isems[0])
    gat_issue(0, 0, gsems[0])
    idx_issue(2, isems[0])

    def body(k, carry):
        even = lax.rem(k, 2) == 0

        def stage(par):
            @pl.when(k + 1 < NCHUNK)
            def _():
                idx_drain(isems[1 - par])
                @pl.when(k >= 1)
                def _():
                    scat_drain(1 - par, ssems[1 - par])
                gat_issue(k + 1, 1 - par, gsems[1 - par])

            @pl.when(k + 3 < NCHUNK)
            def _():
                idx_issue(k + 3, isems[1 - par])

            gat_drain(par, gsems[par])
            scat_issue(k, par, ssems[par])

        @pl.when(even)
        def _():
            stage(0)

        @pl.when(jnp.logical_not(even))
        def _():
            stage(1)

        return carry

    lax.fori_loop(0, NCHUNK, body, 0)
    scat_drain(0, ssems[0])
    scat_drain(1, ssems[1])
    plsc.subcore_barrier()
    pltpu.sync_copy(acc_sh.at[pl.ds(base, RPT)], out_hbm.at[cid, pl.ds(base, RPT)])


R = 400
GRID = N // R


def _row_spec(width):
    return pl.BlockSpec((R, width), lambda i: (i, 0))


def _full_spec(a, b):
    return pl.BlockSpec((a, b), lambda i: (0, 0))


def _enc_body(desc, tweet, nump, catp, wd, wt, wn, wc, bd, bt, bn, bc,
              wid_, wit_, win_, wic_, bin_, out):
    d = _lrelu(jnp.dot(desc[...], wd[...], preferred_element_type=jnp.float32) + bd[...])
    t = _lrelu(jnp.dot(tweet[...], wt[...], preferred_element_type=jnp.float32) + bt[...])
    n = _lrelu(jnp.dot(nump[...], wn[...], preferred_element_type=jnp.float32) + bn[...])
    c = _lrelu(jnp.dot(catp[...], wc[...], preferred_element_type=jnp.float32) + bc[...])
    acc = (jnp.dot(d, wid_[...], preferred_element_type=jnp.float32)
           + jnp.dot(t, wit_[...], preferred_element_type=jnp.float32)
           + jnp.dot(n, win_[...], preferred_element_type=jnp.float32)
           + jnp.dot(c, wic_[...], preferred_element_type=jnp.float32))
    out[...] = _lrelu(acc + bin_[...])


def _dis(deg0, deg1):
    return lax.rsqrt(1.0 + deg0[:, 0:1] + deg1[:, 0:1])


def _h1_body(x0, deg0, deg1, w1t, h1):
    dis = _dis(deg0[...], deg1[...])
    h1[...] = jnp.dot(x0[...], w1t[...], preferred_element_type=jnp.float32) * dis


def _mid_body(p0, p1, hprev, deg0, deg1, bprev, w2t, hnext):
    dis = _dis(deg0[...], deg1[...])
    x1 = dis * (p0[...] + p1[...] + hprev[...]) + bprev[...]
    hnext[...] = jnp.dot(x1, w2t[...], preferred_element_type=jnp.float32) * dis


def _tail_body(p0, p1, hprev, deg0, deg1, bprev, wot, bo, wht, bh, out):
    dis = _dis(deg0[...], deg1[...])
    x2 = dis * (p0[...] + p1[...] + hprev[...]) + bprev[...]
    y = _lrelu(jnp.dot(x2, wot[...], preferred_element_type=jnp.float32) + bo[...])
    out[...] = jnp.dot(y, wht[...], preferred_element_type=jnp.float32) + bh[...]


def kernel(description, tweet, num_prop, cat_prop, edge_index, W_desc, b_desc,
           W_tweet, b_tweet, W_num, b_num, W_cat, b_cat, W_in, b_in, gcn1_W,
           gcn1_b, gcn2_W, gcn2_b, W_out, b_out, W_head, b_head):
    f32 = jnp.float32
    src = edge_index[0].astype(jnp.int32)
    dst = edge_index[1].astype(jnp.int32)

    npad_e = EPAD - E
    spread = jnp.arange(npad_e, dtype=jnp.int32) % (NPAD - N)
    src3 = jnp.concatenate([src, spread]).reshape(NW, NCHUNK, CHUNK)
    dst3 = jnp.concatenate([dst, N + spread]).reshape(NW, NCHUNK, CHUNK)

    deg_parts = _deg_call(dst3)
    deg0, deg1 = deg_parts[0, :N, :16], deg_parts[1, :N, :16]

    enc = pl.pallas_call(
        _enc_body,
        grid=(GRID,),
        in_specs=[
            _row_spec(768), _row_spec(768), _row_spec(5), _row_spec(3),
            _full_spec(768, B), _full_spec(768, B), _full_spec(5, B), _full_spec(3, B),
            _full_spec(1, B), _full_spec(1, B), _full_spec(1, B), _full_spec(1, B),
            _full_spec(B, H), _full_spec(B, H), _full_spec(B, H), _full_spec(B, H),
            _full_spec(1, H),
        ],
        out_specs=_row_spec(H),
        out_shape=jax.ShapeDtypeStruct((N, H), f32),
    )
    x0 = enc(description, tweet, num_prop, cat_prop,
             W_desc.T, W_tweet.T, W_num.T, W_cat.T,
             b_desc.reshape(1, B), b_tweet.reshape(1, B),
             b_num.reshape(1, B), b_cat.reshape(1, B),
             W_in[:, 0:B].T, W_in[:, B:2 * B].T, W_in[:, 2 * B:3 * B].T,
             W_in[:, 3 * B:4 * B].T, b_in.reshape(1, H))

    h1 = pl.pallas_call(
        _h1_body,
        grid=(GRID,),
        in_specs=[_row_spec(H), _row_spec(16), _row_spec(16), _full_spec(H, H)],
        out_specs=_row_spec(H),
        out_shape=jax.ShapeDtypeStruct((N, H), f32),
    )(x0, deg0, deg1, gcn1_W.T)

    s1 = _seg_call(h1, src3, dst3)

    h2 = pl.pallas_call(
        _mid_body,
        grid=(GRID,),
        in_specs=[_row_spec(H), _row_spec(H), _row_spec(H), _row_spec(16),
                  _row_spec(16), _full_spec(1, H), _full_spec(H, H)],
        out_specs=_row_spec(H),
        out_shape=jax.ShapeDtypeStruct((N, H), f32),
    )(s1[0, :N], s1[1, :N], h1, deg0, deg1, gcn1_b.reshape(1, H), gcn2_W.T)

    s2 = _seg_call(h2, src3, dst3)

    out = pl.pallas_call(
        _tail_body,
        grid=(GRID,),
        in_specs=[_row_spec(H), _row_spec(H), _row_spec(H), _row_spec(16),
                  _row_spec(16), _full_spec(1, H), _full_spec(H, H),
                  _full_spec(1, H), _full_spec(H, 2), _full_spec(1, 2)],
        out_specs=_row_spec(2),
        out_shape=jax.ShapeDtypeStruct((N, 2), f32),
    )(s2[0, :N], s2[1, :N], h2, deg0, deg1, gcn2_b.reshape(1, H), W_out.T,
      b_out.reshape(1, H), W_head.T, b_head.reshape(1, 2))

    return out
